# Initial kernel scaffold; baseline (speedup 1.0000x reference)
import functools

import jax
import jax.numpy as jnp
from jax import lax
from jax.experimental import pallas as pl
from jax.experimental.pallas import tpu as pltpu
from jax.experimental.pallas import tpu_sc as plsc

_N = 10000
_E = 160000
_D = 256
_H = 128
_NC = 2
_NS = 16
_EPT = _E // _NS
_CE = 80
_NCHUNK = _EPT // _CE
_WB = 40
_NWB = _N // _WB
_WB_ITERS = (_NWB + _NS - 1) // _NS


def _proj_body(h_ref, w_ref, o_ref):
    h = jnp.concatenate([h_ref[0], h_ref[1]], axis=1)
    for j in range(8):
        o_ref[j] = jnp.dot(h, w_ref[j], preferred_element_type=jnp.float32)


_BN = 400
_proj = pl.pallas_call(
    _proj_body,
    grid=(_N // _BN,),
    in_specs=[
        pl.BlockSpec((2, _BN, _H), lambda i: (0, i, 0)),
        pl.BlockSpec((8, _D, _H), lambda i: (0, 0, 0)),
    ],
    out_specs=pl.BlockSpec((8, _BN, _H), lambda i: (0, i, 0)),
    out_shape=jax.ShapeDtypeStruct((8, _N, _H), jnp.float32),
)


def _edge_body(table, edges, skip, bias, out,
               src_v, dst_v, ik, iq, iv, kd, qs, vs, wb, wb2, bvec, agg, sem):
    c = lax.axis_index("c")
    s = lax.axis_index("s")
    zero16 = jnp.zeros((16,), jnp.float32)

    def _zb(i, _):
        r = i // (_H // 16)
        f = i % (_H // 16)
        wb[r, pl.ds(f * 16, 16)] = zero16
        return 0
    lax.fori_loop(0, _WB * (_H // 16), _zb, 0)
    for k in range(_WB_ITERS):
        cid = s + _NS * k
        @pl.when(cid < _NWB)
        def _():
            pltpu.sync_copy(wb, agg.at[pl.ds(cid * _WB, _WB)])
    plsc.subcore_barrier()

    kbase = c * _N
    qbase = (2 + c) * _N
    vbase = (4 + c) * _N

    def _chunk(k, _):
        base = s * _EPT + k * _CE
        pltpu.sync_copy(edges.at[0, pl.ds(base, _CE)], src_v.at[0])
        pltpu.sync_copy(edges.at[1, pl.ds(base, _CE)], dst_v.at[0])
        for j in range(_CE // 16):
            sl = pl.ds(j * 16, 16)
            sv = src_v[0, sl]
            dv = dst_v[0, sl]
            ik[0, sl] = dv + kbase
            iq[0, sl] = sv + qbase
            iv[0, sl] = sv + vbase
        cp1 = pltpu.async_copy(table.at[ik.at[0]], kd, sem)
        cp2 = pltpu.async_copy(table.at[iq.at[0]], qs, sem)
        cp3 = pltpu.async_copy(table.at[iv.at[0]], vs, sem)
        cp1.wait()
        cp2.wait()
        cp3.wait()

        def _edge(e, _):
            for f in range(_H // 16):
                sl = pl.ds(f * 16, 16)
                z = kd[e, sl] + qs[e, sl]
                kd[e, sl] = vs[e, sl] / (1.0 + jnp.exp(-z))
            return 0
        lax.fori_loop(0, _CE, _edge, 0)
        pltpu.sync_copy(kd, agg.at[dst_v.at[0]], add=True)
        return 0
    lax.fori_loop(0, _NCHUNK, _chunk, 0)
    plsc.subcore_barrier()

    pltpu.sync_copy(bias.at[c], bvec)
    for k in range(_WB_ITERS):
        cid = s + _NS * k
        @pl.when(cid < _NWB)
        def _():
            r0 = cid * _WB
            pltpu.sync_copy(agg.at[pl.ds(r0, _WB)], wb)
            pltpu.sync_copy(skip.at[c, pl.ds(r0, _WB)], wb2)

            def _wb(i, _):
                r = i // (_H // 16)
                f = i % (_H // 16)
                sl = pl.ds(f * 16, 16)
                v = wb[r, sl] + wb2[r, sl] + bvec[sl]
                wb[r, sl] = jnp.maximum(v, 0.0)
                return 0
            lax.fori_loop(0, _WB * (_H // 16), _wb, 0)
            pltpu.sync_copy(wb, out.at[c, pl.ds(r0, _WB)])


_edge_call = functools.partial(
    pl.kernel,
    out_type=jax.ShapeDtypeStruct((_NC, _N, _H), jnp.float32),
    mesh=plsc.VectorSubcoreMesh(core_axis_name="c", subcore_axis_name="s"),
    scratch_types=[
        pltpu.VMEM((1, _CE), jnp.int32),
        pltpu.VMEM((1, _CE), jnp.int32),
        pltpu.VMEM((1, _CE), jnp.int32),
        pltpu.VMEM((1, _CE), jnp.int32),
        pltpu.VMEM((1, _CE), jnp.int32),
        pltpu.VMEM((_CE, _H), jnp.float32),
        pltpu.VMEM((_CE, _H), jnp.float32),
        pltpu.VMEM((_CE, _H), jnp.float32),
        pltpu.VMEM((_WB, _H), jnp.float32),
        pltpu.VMEM((_WB, _H), jnp.float32),
        pltpu.VMEM((_H,), jnp.float32),
        pltpu.VMEM_SHARED((_N, _H), jnp.float32),
        pltpu.SemaphoreType.DMA,
    ],
)(_edge_body)


def _layer(h2, edge_index, Wk, Wq, Wv, Ws, b):
    w = jnp.concatenate([Wk, Wq, Wv, Ws], axis=1)
    wt = w.reshape(_D, 8, _H).transpose(1, 0, 2)
    proj = _proj(h2, wt)
    table = proj[:6].reshape(6 * _N, _H)
    skip = proj[6:]
    return _edge_call(table, edge_index, skip, b.reshape(_NC, _H))


def kernel(x, edge_index, Wk0, Wq0, Wv0, Ws0, Wk1, Wq1, Wv1, Ws1, b0, b1):
    h2 = x.reshape(_N, _NC, _H).transpose(1, 0, 2)
    h2 = _layer(h2, edge_index, Wk0, Wq0, Wv0, Ws0, b0)
    h2 = _layer(h2, edge_index, Wk1, Wq1, Wv1, Ws1, b1)
    return jnp.concatenate([h2[0], h2[1]], axis=1)


# speedup vs baseline: 2.7403x; 2.7403x over previous
import functools

import jax
import jax.numpy as jnp
from jax import lax
from jax.experimental import pallas as pl
from jax.experimental.pallas import tpu as pltpu
from jax.experimental.pallas import tpu_sc as plsc

_N = 10000
_E = 160000
_D = 256
_H = 128
_NC = 2
_NS = 16
_EPT = _E // _NS
_CE = 80
_NCHUNK = _EPT // _CE
_WB = 40
_NWB = _N // _WB
_WB_ITERS = (_NWB + _NS - 1) // _NS


def _proj_body(h_ref, w_ref, o_ref):
    h = jnp.concatenate([h_ref[0], h_ref[1]], axis=1)
    for j in range(8):
        o_ref[j] = jnp.dot(h, w_ref[j], preferred_element_type=jnp.float32)


_BN = 400
_proj = pl.pallas_call(
    _proj_body,
    grid=(_N // _BN,),
    in_specs=[
        pl.BlockSpec((2, _BN, _H), lambda i: (0, i, 0)),
        pl.BlockSpec((8, _D, _H), lambda i: (0, 0, 0)),
    ],
    out_specs=pl.BlockSpec((8, _BN, _H), lambda i: (0, i, 0)),
    out_shape=jax.ShapeDtypeStruct((8, _N, _H), jnp.float32),
)


def _edge_body(table, src_hbm, dst_hbm, skip, bias, out,
               src_v, dst_v, ik, iq, iv, kd, qs, vs, wb, wb2, bvec, agg, sem):
    c = lax.axis_index("c")
    s = lax.axis_index("s")
    zero16 = jnp.zeros((16,), jnp.float32)

    def _zb(i, _):
        r = i // (_H // 16)
        f = i % (_H // 16)
        wb[r, pl.ds(f * 16, 16)] = zero16
        return 0
    lax.fori_loop(0, _WB * (_H // 16), _zb, 0)
    for k in range(_WB_ITERS):
        cid = s + _NS * k
        @pl.when(cid < _NWB)
        def _():
            pltpu.sync_copy(wb, agg.at[pl.ds(cid * _WB, _WB)])
    plsc.subcore_barrier()

    kbase = c * _N
    qbase = (2 + c) * _N
    vbase = (4 + c) * _N

    def _chunk(k, _):
        base = s * _EPT + k * _CE
        pltpu.sync_copy(src_hbm.at[pl.ds(base, _CE)], src_v.at[0])
        pltpu.sync_copy(dst_hbm.at[pl.ds(base, _CE)], dst_v.at[0])
        for j in range(_CE // 16):
            sl = pl.ds(j * 16, 16)
            sv = src_v[0, sl]
            dv = dst_v[0, sl]
            ik[0, sl] = dv + kbase
            iq[0, sl] = sv + qbase
            iv[0, sl] = sv + vbase
        cp1 = pltpu.async_copy(table.at[ik.at[0]], kd, sem)
        cp2 = pltpu.async_copy(table.at[iq.at[0]], qs, sem)
        cp3 = pltpu.async_copy(table.at[iv.at[0]], vs, sem)
        cp1.wait()
        cp2.wait()
        cp3.wait()

        def _edge(e, _):
            for f in range(_H // 16):
                sl = pl.ds(f * 16, 16)
                z = kd[e, sl] + qs[e, sl]
                kd[e, sl] = vs[e, sl] / (1.0 + jnp.exp(-z))
            return 0
        lax.fori_loop(0, _CE, _edge, 0)
        pltpu.sync_copy(kd, agg.at[dst_v.at[0]], add=True)
        return 0
    lax.fori_loop(0, _NCHUNK, _chunk, 0)
    plsc.subcore_barrier()

    pltpu.sync_copy(bias.at[c], bvec)
    for k in range(_WB_ITERS):
        cid = s + _NS * k
        @pl.when(cid < _NWB)
        def _():
            r0 = cid * _WB
            pltpu.sync_copy(agg.at[pl.ds(r0, _WB)], wb)
            pltpu.sync_copy(skip.at[c, pl.ds(r0, _WB)], wb2)

            def _wb(i, _):
                r = i // (_H // 16)
                f = i % (_H // 16)
                sl = pl.ds(f * 16, 16)
                v = wb[r, sl] + wb2[r, sl] + bvec[sl]
                wb[r, sl] = jnp.maximum(v, 0.0)
                return 0
            lax.fori_loop(0, _WB * (_H // 16), _wb, 0)
            pltpu.sync_copy(wb, out.at[c, pl.ds(r0, _WB)])


_edge_call = functools.partial(
    pl.kernel,
    out_type=jax.ShapeDtypeStruct((_NC, _N, _H), jnp.float32),
    mesh=plsc.VectorSubcoreMesh(core_axis_name="c", subcore_axis_name="s"),
    scratch_types=[
        pltpu.VMEM((1, _CE), jnp.int32),
        pltpu.VMEM((1, _CE), jnp.int32),
        pltpu.VMEM((1, _CE), jnp.int32),
        pltpu.VMEM((1, _CE), jnp.int32),
        pltpu.VMEM((1, _CE), jnp.int32),
        pltpu.VMEM((_CE, _H), jnp.float32),
        pltpu.VMEM((_CE, _H), jnp.float32),
        pltpu.VMEM((_CE, _H), jnp.float32),
        pltpu.VMEM((_WB, _H), jnp.float32),
        pltpu.VMEM((_WB, _H), jnp.float32),
        pltpu.VMEM((_H,), jnp.float32),
        pltpu.VMEM_SHARED((_N, _H), jnp.float32),
        pltpu.SemaphoreType.DMA,
    ],
)(_edge_body)


def _layer(h2, src, dst, Wk, Wq, Wv, Ws, b):
    w = jnp.concatenate([Wk, Wq, Wv, Ws], axis=1)
    wt = w.reshape(_D, 8, _H).transpose(1, 0, 2)
    proj = _proj(h2, wt)
    table = proj[:6].reshape(6 * _N, _H)
    skip = proj[6:]
    return _edge_call(table, src, dst, skip, b.reshape(_NC, _H))


def kernel(x, edge_index, Wk0, Wq0, Wv0, Ws0, Wk1, Wq1, Wv1, Ws1, b0, b1):
    src = edge_index[0]
    dst = edge_index[1]
    h2 = x.reshape(_N, _NC, _H).transpose(1, 0, 2)
    h2 = _layer(h2, src, dst, Wk0, Wq0, Wv0, Ws0, b0)
    h2 = _layer(h2, src, dst, Wk1, Wq1, Wv1, Ws1, b1)
    return jnp.concatenate([h2[0], h2[1]], axis=1)
